# baseline (speedup 1.0000x reference)
import functools
import math

import jax
import jax.numpy as jnp
from jax.experimental import pallas as pl
from jax.experimental.pallas import tpu as pltpu

_ANCHOR_RATIO = 0.1
_MIN_ANCHORS = 1


def _sumsq_d_sublane(yt):
    t = yt[:, 0:8, :]
    for a in range(1, 8):
        t = t + yt[:, 8 * a:8 * a + 8, :]
    u = t[:, 0:4, :] + t[:, 4:8, :]
    v = u[:, 0:2, :] + u[:, 2:4, :]
    return v[:, 0, :] + v[:, 1, :]


def _anchor_body(patches_ref, adp_ref, out_ref, acc_scores, acc_meant,
                 *, k, kpad, n, pb):
    j = pl.program_id(1)
    nj = pl.num_programs(1)

    x = patches_ref[0]
    xt = jnp.transpose(x, (0, 2, 1))
    imp = jnp.mean(adp_ref[...], axis=0)
    norms = jnp.sqrt(_sumsq_d_sublane(xt * xt))
    acc_scores[0, pl.ds(j * pb, pb)] = jax.lax.dot_general(
        imp[None, :], norms, (((1,), (0,)), ((), ())),
        preferred_element_type=jnp.float32)[0]
    acc_meant[:, pl.ds(j * pb, pb)] = jnp.sum(xt, axis=0) * (1.0 / n)

    @pl.when(j == nj - 1)
    def _finalize():
        scores = acc_scores[...]
        p = scores.shape[1]
        meant = acc_meant[...]
        srow = scores
        scol = scores.reshape(p, 1)
        ii = jax.lax.broadcasted_iota(jnp.int32, (p, p), 0)
        jj = jax.lax.broadcasted_iota(jnp.int32, (p, p), 1)
        beats = (scol > srow) | ((scol == srow) & (ii < jj))
        rank = jnp.sum(beats.astype(jnp.int32), axis=0, keepdims=True)
        kk = jax.lax.broadcasted_iota(jnp.int32, (kpad, p), 0)
        onehot = (kk == rank).astype(jnp.float32)
        anchors_t = jax.lax.dot_general(
            meant, onehot, (((1,), (1,)), ((), ())),
            precision=jax.lax.Precision.HIGHEST,
            preferred_element_type=jnp.float32)
        anchors = jnp.transpose(anchors_t, (1, 0))
        out_ref[0] = jnp.broadcast_to(anchors[None, :k, :], out_ref.shape[1:])


def kernel(patches, adp):
    b, n, p, d = patches.shape
    if p == 0:
        return jnp.zeros((b * n, 0, d), dtype=patches.dtype)
    k = max(_MIN_ANCHORS, int(math.ceil(p * _ANCHOR_RATIO)))
    k = min(k, p)
    kpad = max(8, ((k + 7) // 8) * 8)
    pb = 128
    while p % pb:
        pb //= 2

    body = functools.partial(_anchor_body, k=k, kpad=kpad, n=n, pb=pb)
    out = pl.pallas_call(
        body,
        grid=(b, p // pb),
        in_specs=[
            pl.BlockSpec((1, n, pb, d), lambda i, j: (i, 0, j, 0)),
            pl.BlockSpec(adp.shape, lambda i, j: (0, 0)),
        ],
        out_specs=pl.BlockSpec((1, n, k, d), lambda i, j: (i, 0, 0, 0)),
        out_shape=jax.ShapeDtypeStruct((b, n, k, d), patches.dtype),
        scratch_shapes=[
            pltpu.VMEM((1, p), jnp.float32),
            pltpu.VMEM((d, p), jnp.float32),
        ],
    )(patches, adp)
    return out.reshape(b * n, k, d)


# speedup vs baseline: 1.0207x; 1.0207x over previous
import functools
import math

import jax
import jax.numpy as jnp
from jax.experimental import pallas as pl
from jax.experimental.pallas import tpu as pltpu

_ANCHOR_RATIO = 0.1
_MIN_ANCHORS = 1


def _sumsq_d_sublane(yt):
    t = yt[:, 0:8, :]
    for a in range(1, 8):
        t = t + yt[:, 8 * a:8 * a + 8, :]
    u = t[:, 0:4, :] + t[:, 4:8, :]
    v = u[:, 0:2, :] + u[:, 2:4, :]
    return v[:, 0, :] + v[:, 1, :]


def _stream_body(patches_ref, adp_ref, scores_ref, meant_ref, *, n):
    x = patches_ref[0]
    xt = jnp.transpose(x, (0, 2, 1))
    imp = jnp.mean(adp_ref[...], axis=0)
    norms = jnp.sqrt(_sumsq_d_sublane(xt * xt))
    scores_ref[0] = jax.lax.dot_general(
        imp[None, :], norms, (((1,), (0,)), ((), ())),
        preferred_element_type=jnp.float32)
    meant_ref[0] = jnp.sum(xt, axis=0) * (1.0 / n)


def _select_body(scores_ref, meant_ref, out_ref, *, k, kpad, n, d):
    scores = scores_ref[0]
    p = scores.shape[1]
    meant = meant_ref[0]
    srow = scores
    scol = scores.reshape(p, 1)
    ii = jax.lax.broadcasted_iota(jnp.int32, (p, p), 0)
    jj = jax.lax.broadcasted_iota(jnp.int32, (p, p), 1)
    beats = (scol > srow) | ((scol == srow) & (ii < jj))
    rank = jnp.sum(beats.astype(jnp.int32), axis=0, keepdims=True)
    kk = jax.lax.broadcasted_iota(jnp.int32, (kpad, p), 0)
    onehot = (kk == rank).astype(jnp.float32)
    anchors_t = jax.lax.dot_general(
        meant, onehot, (((1,), (1,)), ((), ())),
        precision=jax.lax.Precision.HIGHEST,
        preferred_element_type=jnp.float32)
    anchors = jnp.transpose(anchors_t, (1, 0))
    out_ref[0] = jnp.broadcast_to(anchors[None, :k, :], out_ref.shape[1:])


def kernel(patches, adp):
    b, n, p, d = patches.shape
    if p == 0:
        return jnp.zeros((b * n, 0, d), dtype=patches.dtype)
    k = max(_MIN_ANCHORS, int(math.ceil(p * _ANCHOR_RATIO)))
    k = min(k, p)
    kpad = max(8, ((k + 7) // 8) * 8)
    pb = 128
    while p % pb:
        pb //= 2

    stream = functools.partial(_stream_body, n=n)
    scores, meant = pl.pallas_call(
        stream,
        grid=(b, p // pb),
        in_specs=[
            pl.BlockSpec((1, n, pb, d), lambda i, j: (i, 0, j, 0)),
            pl.BlockSpec(adp.shape, lambda i, j: (0, 0)),
        ],
        out_specs=[
            pl.BlockSpec((1, 1, pb), lambda i, j: (i, 0, j)),
            pl.BlockSpec((1, d, pb), lambda i, j: (i, 0, j)),
        ],
        out_shape=[
            jax.ShapeDtypeStruct((b, 1, p), jnp.float32),
            jax.ShapeDtypeStruct((b, d, p), jnp.float32),
        ],
    )(patches, adp)

    select = functools.partial(_select_body, k=k, kpad=kpad, n=n, d=d)
    out = pl.pallas_call(
        select,
        grid=(b,),
        in_specs=[
            pl.BlockSpec((1, 1, p), lambda i: (i, 0, 0)),
            pl.BlockSpec((1, d, p), lambda i: (i, 0, 0)),
        ],
        out_specs=pl.BlockSpec((1, n, k, d), lambda i: (i, 0, 0, 0)),
        out_shape=jax.ShapeDtypeStruct((b, n, k, d), patches.dtype),
    )(scores, meant)
    return out.reshape(b * n, k, d)
